# baseline (speedup 1.0000x reference)
import functools

import jax
import jax.numpy as jnp
from jax import lax
from jax.experimental import pallas as pl
from jax.experimental.pallas import tpu as pltpu
from jax.experimental.pallas import tpu_sc as plsc


def kernel(input, indices):
    X, Y, N = input.shape
    K = indices.shape[0]
    NC, NS = 2, 16
    NW = NC * NS
    CC = 8
    NCHUNK = K // CC
    UNITS = Y * NCHUNK
    ITERS = -(-UNITS // NW)
    NBUF = 3

    t = jnp.transpose(input, (1, 2, 0))

    mesh = plsc.VectorSubcoreMesh(core_axis_name="c", subcore_axis_name="s")

    @functools.partial(
        pl.kernel,
        mesh=mesh,
        compiler_params=pltpu.CompilerParams(
            needs_layout_passes=False, use_tc_tiling_on_sc=True),
        out_type=jax.ShapeDtypeStruct((Y, K, X), jnp.float32),
        scratch_types=[
            pltpu.VMEM((K,), jnp.int32),
            pltpu.VMEM((NBUF, CC, X), jnp.float32),
            pltpu.SemaphoreType.DMA((NBUF,)),
            pltpu.SemaphoreType.DMA((NBUF,)),
        ],
    )
    def sc_run(in_hbm, idx_hbm, out_hbm, idx_v, buf_v, in_sems, out_sems):
        w = lax.axis_index("s") * NC + lax.axis_index("c")
        pltpu.sync_copy(idx_hbm, idx_v)

        def unit_jc(i):
            unit = w + i * NW
            return unit // NCHUNK, lax.rem(unit, NCHUNK) * CC

        def gather(i, b):
            j, c0 = unit_jc(i)
            return pltpu.make_async_copy(
                in_hbm.at[j].at[idx_v.at[pl.ds(pl.multiple_of(c0, 8), CC)]],
                buf_v.at[b],
                in_sems.at[b])

        def out_copy(i, b):
            j, c0 = unit_jc(i)
            return pltpu.make_async_copy(
                buf_v.at[b],
                out_hbm.at[j, pl.ds(pl.multiple_of(c0, 8), CC)],
                out_sems.at[b])

        def guarded(i, fn):
            @pl.when(w + i * NW < UNITS)
            def _():
                fn()

        for u in range(2):
            guarded(u, lambda u=u: gather(u, u % NBUF).start())

        @pl.loop(0, ITERS)
        def _(u):
            b = lax.rem(u, NBUF)
            guarded(u, lambda: gather(u, b).wait())
            guarded(u, lambda: out_copy(u, b).start())

            @pl.when(u + 2 < ITERS)
            def _():
                bn = lax.rem(u + 2, NBUF)

                @pl.when(u >= 1)
                def _():
                    guarded(u - 1, lambda: out_copy(u - 1, bn).wait())

                guarded(u + 2, lambda: gather(u + 2, bn).start())

        for u in range(ITERS - NBUF, ITERS):
            guarded(u, lambda u=u: out_copy(u, u % NBUF).wait())

    out_t = sc_run(t, indices)
    return jnp.transpose(out_t, (2, 0, 1))
